# baseline (device time: 35893 ns/iter reference)
import jax
import jax.numpy as jnp
from jax import lax
from jax.experimental import pallas as pl
from jax.experimental.pallas import tpu as pltpu

K = 16


def kernel(x):
    m_per, n = x.shape
    half = m_per // 2
    cr = half // K

    def body(x_ref, out_ref, ysend, yrecv, xsend, xrecv, own_sem):
        my_x = lax.axis_index("x")
        my_y = lax.axis_index("y")
        y_nbr = (my_x, 1 - my_y)
        x_nbr = (1 - my_x, my_y)

        send_base = my_y * m_per + my_x * half
        recv_base = (1 - my_y) * m_per + my_x * half

        barrier_sem = pltpu.get_barrier_semaphore()
        for nbr in (y_nbr, x_nbr):
            pl.semaphore_signal(
                barrier_sem, inc=1,
                device_id=nbr, device_id_type=pl.DeviceIdType.MESH,
            )
        pl.semaphore_wait(barrier_sem, 2)

        y_rdmas = []
        for c in range(K):
            rdma = pltpu.make_async_remote_copy(
                src_ref=x_ref.at[pl.ds(my_x * half + c * cr, cr), :],
                dst_ref=out_ref.at[pl.ds(send_base + c * cr, cr), :],
                send_sem=ysend.at[c],
                recv_sem=yrecv.at[c],
                device_id=y_nbr,
                device_id_type=pl.DeviceIdType.MESH,
            )
            rdma.start()
            y_rdmas.append(rdma)

        own = pltpu.make_async_copy(
            x_ref, out_ref.at[pl.ds(my_y * m_per, m_per), :], own_sem
        )
        own.start()

        x_rdmas = []
        for c in range(K):
            y_rdmas[c].wait_recv()
            rdma = pltpu.make_async_remote_copy(
                src_ref=out_ref.at[pl.ds(recv_base + c * cr, cr), :],
                dst_ref=out_ref.at[pl.ds(recv_base + c * cr, cr), :],
                send_sem=xsend.at[c],
                recv_sem=xrecv.at[c],
                device_id=x_nbr,
                device_id_type=pl.DeviceIdType.MESH,
            )
            rdma.start()
            x_rdmas.append(rdma)

        for c in range(K):
            x_rdmas[c].wait_recv()
        own.wait()
        for c in range(K):
            y_rdmas[c].wait_send()
            x_rdmas[c].wait_send()

    return pl.pallas_call(
        body,
        out_shape=jax.ShapeDtypeStruct((2 * m_per, n), x.dtype),
        in_specs=[pl.BlockSpec(memory_space=pltpu.MemorySpace.HBM)],
        out_specs=pl.BlockSpec(memory_space=pltpu.MemorySpace.HBM),
        scratch_shapes=[
            pltpu.SemaphoreType.DMA((K,)),
            pltpu.SemaphoreType.DMA((K,)),
            pltpu.SemaphoreType.DMA((K,)),
            pltpu.SemaphoreType.DMA((K,)),
            pltpu.SemaphoreType.DMA,
        ],
        compiler_params=pltpu.CompilerParams(collective_id=0),
    )(x)


# device time: 34856 ns/iter; 1.0298x vs baseline; 1.0298x over previous
import jax
import jax.numpy as jnp
from jax import lax
from jax.experimental import pallas as pl
from jax.experimental.pallas import tpu as pltpu

M = 2048
P = 1088
F = M - P
R = P - F
CR = 64
KF = F // CR
KR = R // CR
KY = KF + KR


def kernel(x):
    m_per, n = x.shape
    assert m_per == M

    def body(x_ref, out_ref, fbuf, ysend, yrecv, xsend, xrecv, fcopy, own_sem):
        my_x = lax.axis_index("x")
        my_y = lax.axis_index("y")
        y_nbr = (my_x, 1 - my_y)
        x_nbr = (1 - my_x, my_y)

        f_base = my_x * P
        other = (1 - my_y) * M

        barrier_sem = pltpu.get_barrier_semaphore()
        for nbr in (y_nbr, x_nbr):
            pl.semaphore_signal(
                barrier_sem, inc=1,
                device_id=nbr, device_id_type=pl.DeviceIdType.MESH,
            )
        pl.semaphore_wait(barrier_sem, 2)

        y_rdmas = []
        for c in range(KF):
            rdma = pltpu.make_async_remote_copy(
                src_ref=x_ref.at[pl.ds(f_base + c * CR, CR), :],
                dst_ref=fbuf.at[c],
                send_sem=ysend.at[c],
                recv_sem=yrecv.at[c],
                device_id=y_nbr,
                device_id_type=pl.DeviceIdType.MESH,
            )
            rdma.start()
            y_rdmas.append(rdma)
        for c in range(KR):
            rdma = pltpu.make_async_remote_copy(
                src_ref=x_ref.at[pl.ds(F + c * CR, CR), :],
                dst_ref=out_ref.at[pl.ds(my_y * M + F + c * CR, CR), :],
                send_sem=ysend.at[KF + c],
                recv_sem=yrecv.at[KF + c],
                device_id=y_nbr,
                device_id_type=pl.DeviceIdType.MESH,
            )
            rdma.start()
            y_rdmas.append(rdma)

        own = pltpu.make_async_copy(
            x_ref, out_ref.at[pl.ds(my_y * M, M), :], own_sem
        )
        own.start()

        x_rdmas = []
        copies = []
        for c in range(KF):
            y_rdmas[c].wait_recv()
            rdma = pltpu.make_async_remote_copy(
                src_ref=fbuf.at[c],
                dst_ref=out_ref.at[pl.ds(other + f_base + c * CR, CR), :],
                send_sem=xsend.at[c],
                recv_sem=xrecv.at[c],
                device_id=x_nbr,
                device_id_type=pl.DeviceIdType.MESH,
            )
            rdma.start()
            x_rdmas.append(rdma)
            cp = pltpu.make_async_copy(
                fbuf.at[c],
                out_ref.at[pl.ds(other + f_base + c * CR, CR), :],
                fcopy.at[c],
            )
            cp.start()
            copies.append(cp)

        for c in range(KR):
            y_rdmas[KF + c].wait_recv()
        for c in range(KF):
            x_rdmas[c].wait_recv()
        own.wait()
        for c in range(KF):
            copies[c].wait()
        for c in range(KY):
            y_rdmas[c].wait_send()
        for c in range(KF):
            x_rdmas[c].wait_send()

    return pl.pallas_call(
        body,
        out_shape=jax.ShapeDtypeStruct((2 * m_per, n), x.dtype),
        in_specs=[pl.BlockSpec(memory_space=pltpu.MemorySpace.HBM)],
        out_specs=pl.BlockSpec(memory_space=pltpu.MemorySpace.HBM),
        scratch_shapes=[
            pltpu.VMEM((KF, CR, n), x.dtype),
            pltpu.SemaphoreType.DMA((KY,)),
            pltpu.SemaphoreType.DMA((KY,)),
            pltpu.SemaphoreType.DMA((KF,)),
            pltpu.SemaphoreType.DMA((KF,)),
            pltpu.SemaphoreType.DMA((KF,)),
            pltpu.SemaphoreType.DMA,
        ],
        compiler_params=pltpu.CompilerParams(collective_id=0),
    )(x)
